# baseline (speedup 1.0000x reference)
import functools

import jax
import jax.numpy as jnp
from jax import lax
from jax.experimental import pallas as pl
from jax.experimental.pallas import tpu as pltpu
from jax.experimental.pallas import tpu_sc as plsc

_NC = 2
_NS = 16
_NW = _NC * _NS
_L = 16
_TBL = 64
_CHT = 16


def _make_sc_encode(n, C):
    TRC = -(-C // 8)
    NT = -(-n // 128)
    CH = _CHT * 128
    G = CH // _L
    BW = _CHT * 1024
    NCHUNK = -(-NT // _CHT)
    MAXC = -(-NCHUNK // _NW)

    LAST_T0 = NT - _CHT
    LAST_R0 = LAST_T0 * 128
    LAST_G = (n - LAST_R0) // _L
    LAST_SRC = n - CH
    assert LAST_R0 - LAST_SRC >= 0 and (n - LAST_R0) % _L == 0
    assert LAST_SRC % 8 == 0

    mesh = plsc.VectorSubcoreMesh(
        core_axis_name="c", subcore_axis_name="s",
        num_cores=_NC, num_subcores=_NS)

    @functools.partial(
        pl.kernel,
        out_type=jax.ShapeDtypeStruct((TRC * NT * 1024,), jnp.float32),
        mesh=mesh,
        compiler_params=pltpu.CompilerParams(
            needs_layout_passes=False, use_tc_tiling_on_sc=False),
        scratch_types=[
            pltpu.VMEM((_L,), jnp.int32),
            pltpu.VMEM((_TBL,), jnp.int32),
            pltpu.VMEM((_TBL,), jnp.float32),
            pltpu.VMEM((_TBL,), jnp.float32),
            pltpu.VMEM((CH,), jnp.int32),
            pltpu.VMEM((CH,), jnp.int32),
            pltpu.VMEM((TRC * BW,), jnp.float32),
            pltpu.VMEM((TRC * BW,), jnp.float32),
            pltpu.VMEM((CH,), jnp.int32),
            pltpu.VMEM((CH,), jnp.int32),
            pltpu.VMEM((CH,), jnp.int32),
            pltpu.VMEM((CH,), jnp.int32),
            pltpu.SemaphoreType.DMA,
            pltpu.SemaphoreType.DMA,
            pltpu.SemaphoreType.DMA,
            pltpu.SemaphoreType.DMA,
            pltpu.SemaphoreType.DMA,
            pltpu.SemaphoreType.DMA,
        ],
    )
    def encode(az_hbm, zs_hbm, out_hbm, zs_v, cpk_t, vlo_t, vhi_t,
               idx0, idx1, outb0, outb1, alo0, alo1, ahi0, ahi1,
               sem00, sem01, sem10, sem11, isem0, isem1):
        wid = lax.axis_index("s") * _NC + lax.axis_index("c")
        outbs = (outb0, outb1)
        alos = (alo0, alo1)
        ahis = (ahi0, ahi1)
        idxbufs = (idx0, idx1)
        isems = (isem0, isem1)
        sems = ((sem00, sem01), (sem10, sem11))

        pltpu.sync_copy(zs_hbm, zs_v.at[pl.ds(0, C)])

        iota = lax.iota(jnp.int32, _L)
        zrow = zs_v[...]
        zk = [zrow[k] for k in range(C)]
        for t in range(_TBL // _L):
            v = iota + (t * _L)
            j = jnp.zeros((_L,), jnp.int32)
            for k in range(C):
                j = j + jnp.where(zk[k] < v, 1, 0).astype(jnp.int32)
            jc = jnp.minimum(j, C - 1)
            lo = jnp.maximum(jc - 1, 0)
            zhi = plsc.load_gather(zs_v, [jc])
            zlo = plsc.load_gather(zs_v, [lo])
            exact = zhi == v
            v_f = v.astype(jnp.float32)
            zhi_f = zhi.astype(jnp.float32)
            zlo_f = zlo.astype(jnp.float32)
            denom = jnp.maximum(zhi_f - zlo_f, 1.0)
            w_lo = (zhi_f - v_f) / denom
            w_hi = (v_f - zlo_f) / denom
            sl = pl.ds(t * _L, _L)
            cpk_t[sl] = jnp.where(exact, jc, lo) + jc * 256
            vlo_t[sl] = jnp.where(exact, 1.0, w_lo)
            vhi_t[sl] = jnp.where(exact, 1.0, w_hi)

        zeros = jnp.zeros((_L,), jnp.float32)
        for b in range(2):
            outb = outbs[b]

            @plsc.parallel_loop(0, TRC * BW // _L, unroll=4)
            def _(i):
                outb[pl.ds(i * _L, _L)] = zeros

        def issue_idx(c):
            cid = c * _NW + wid

            @pl.when(cid < NCHUNK)
            def _():
                src = jnp.where(cid == NCHUNK - 1, LAST_SRC, cid * CH)
                pltpu.async_copy(
                    az_hbm.at[pl.ds(src, CH)], idxbufs[c % 2], isems[c % 2])

        issue_idx(0)
        for c in range(MAXC):
            if c + 1 < MAXC:
                issue_idx(c + 1)
            b = c % 2
            outb = outbs[b]
            alo_v = alos[b]
            ahi_v = ahis[b]
            idx_v = idxbufs[b]
            cid = c * _NW + wid

            @pl.when(cid < NCHUNK)
            def _():
                last = cid == NCHUNK - 1
                goff = jnp.where(last, (LAST_R0 - LAST_SRC) // _L, 0)
                ngrp = jnp.where(last, LAST_G, G)
                t0 = jnp.where(last, LAST_T0, cid * _CHT)
                if c >= 2:
                    for tr in range(TRC):
                        pltpu.make_async_copy(
                            out_hbm.at[pl.ds(0, BW)], outb.at[pl.ds(0, BW)],
                            sems[b][tr]).wait()
                pltpu.make_async_copy(
                    az_hbm.at[pl.ds(0, CH)], idx_v, isems[b]).wait()

                @plsc.parallel_loop(0, ngrp, unroll=2)
                def _(g):
                    sl = pl.ds(g * _L, _L)
                    if c >= 2:
                        plsc.store_scatter(outb, [alo_v[sl]], zeros)
                        plsc.store_scatter(outb, [ahi_v[sl]], zeros)
                    z = idx_v[pl.ds((g + goff) * _L, _L)]
                    z = jnp.minimum(jnp.maximum(z, 0), _TBL - 1)
                    cpk = plsc.load_gather(cpk_t, [z])
                    vlo = plsc.load_gather(vlo_t, [z])
                    vhi = plsc.load_gather(vhi_t, [z])
                    clo = cpk & 255
                    chi = cpk >> 8
                    base = g * _L + iota
                    lanes = (base >> 7) * 1024 + (base & 127)
                    alo = (clo >> 3) * BW + ((clo & 7) << 7) + lanes
                    ahi = (chi >> 3) * BW + ((chi & 7) << 7) + lanes
                    alo_v[sl] = alo
                    ahi_v[sl] = ahi
                    plsc.store_scatter(outb, [alo], vlo)
                    plsc.store_scatter(outb, [ahi], vhi)

                for tr in range(TRC):
                    pltpu.async_copy(
                        outb.at[pl.ds(tr * BW, BW)],
                        out_hbm.at[pl.ds(tr * NT * 1024 + t0 * 1024, BW)],
                        sems[b][tr])

        for b in range(2):
            for tr in range(TRC):
                pltpu.make_async_copy(
                    out_hbm.at[pl.ds(0, BW)], outbs[b].at[pl.ds(0, BW)],
                    sems[b][tr]).wait()

    return encode


def kernel(atomic_numbers, zs):
    n = atomic_numbers.shape[0]
    C = zs.shape[0]
    TRC = -(-C // 8)
    NT = -(-n // 128)
    enc = _make_sc_encode(n, C)
    out_flat = enc(atomic_numbers.astype(jnp.int32), zs.astype(jnp.int32))
    out4 = out_flat.reshape(TRC, NT, 8, 128)
    return out4.transpose(1, 3, 0, 2).reshape(NT * 128, TRC * 8)[:n, :C]


# speedup vs baseline: 1.1164x; 1.1164x over previous
import functools

import jax
import jax.numpy as jnp
from jax import lax
from jax.experimental import pallas as pl
from jax.experimental.pallas import tpu as pltpu
from jax.experimental.pallas import tpu_sc as plsc

_NC = 2
_NS = 16
_NW = _NC * _NS
_L = 16
_TBL = 64
_CHT = 24


def _make_sc_encode(n, C):
    TRC = -(-C // 8)
    NT = -(-n // 128)
    CH = _CHT * 128
    NCHUNK = -(-NT // _CHT)
    MAXC = -(-NCHUNK // _NW)

    LAST_T0 = NT - _CHT
    LAST_R0 = LAST_T0 * 128
    LAST_G = (n - LAST_R0) // _L
    LAST_SRC = n - CH
    assert LAST_R0 - LAST_SRC >= 0 and (n - LAST_R0) % _L == 0
    assert LAST_SRC % 8 == 0

    mesh = plsc.VectorSubcoreMesh(
        core_axis_name="c", subcore_axis_name="s",
        num_cores=_NC, num_subcores=_NS)

    @functools.partial(
        pl.kernel,
        out_type=jax.ShapeDtypeStruct((TRC, NT, 8, 128), jnp.float32),
        mesh=mesh,
        compiler_params=pltpu.CompilerParams(
            needs_layout_passes=False, use_tc_tiling_on_sc=False),
        scratch_types=[
            pltpu.VMEM((_L,), jnp.int32),
            pltpu.VMEM((_TBL,), jnp.int32),
            pltpu.VMEM((_TBL,), jnp.float32),
            pltpu.VMEM((_TBL,), jnp.float32),
            pltpu.VMEM((CH,), jnp.int32),
            pltpu.VMEM((CH,), jnp.int32),
            pltpu.VMEM((TRC, _CHT, 8, 128), jnp.float32),
            pltpu.VMEM((TRC, _CHT, 8, 128), jnp.float32),
            pltpu.SemaphoreType.DMA,
            pltpu.SemaphoreType.DMA,
            pltpu.SemaphoreType.DMA,
            pltpu.SemaphoreType.DMA,
            pltpu.SemaphoreType.DMA,
            pltpu.SemaphoreType.DMA,
        ],
    )
    def encode(az_hbm, zs_hbm, out_hbm, zs_v, cpk_t, vlo_t, vhi_t,
               idx0, idx1, outb0, outb1, sem00, sem01, sem10, sem11,
               isem0, isem1):
        wid = lax.axis_index("s") * _NC + lax.axis_index("c")
        outbs = (outb0, outb1)
        idxbufs = (idx0, idx1)
        isems = (isem0, isem1)
        sems = ((sem00, sem01), (sem10, sem11))

        pltpu.sync_copy(zs_hbm, zs_v.at[pl.ds(0, C)])

        iota = lax.iota(jnp.int32, _L)
        zrow = zs_v[...]
        zk = [zrow[k] for k in range(C)]
        for t in range(_TBL // _L):
            v = iota + (t * _L)
            j = jnp.zeros((_L,), jnp.int32)
            for k in range(C):
                j = j + jnp.where(zk[k] < v, 1, 0).astype(jnp.int32)
            jc = jnp.minimum(j, C - 1)
            lo = jnp.maximum(jc - 1, 0)
            zhi = plsc.load_gather(zs_v, [jc])
            zlo = plsc.load_gather(zs_v, [lo])
            exact = zhi == v
            v_f = v.astype(jnp.float32)
            zhi_f = zhi.astype(jnp.float32)
            zlo_f = zlo.astype(jnp.float32)
            denom = jnp.maximum(zhi_f - zlo_f, 1.0)
            w_lo = (zhi_f - v_f) / denom
            w_hi = (v_f - zlo_f) / denom
            sl = pl.ds(t * _L, _L)
            cpk_t[sl] = jnp.where(exact, jc, lo) + jc * 256
            vlo_t[sl] = jnp.where(exact, 1.0, w_lo)
            vhi_t[sl] = jnp.where(exact, 1.0, w_hi)

        zeros = jnp.zeros((_L,), jnp.float32)

        def issue_idx(c):
            cid = c * _NW + wid

            @pl.when(cid < NCHUNK)
            def _():
                src = jnp.where(cid == NCHUNK - 1, LAST_SRC, cid * CH)
                pltpu.async_copy(
                    az_hbm.at[pl.ds(src, CH)], idxbufs[c % 2], isems[c % 2])

        issue_idx(0)
        for c in range(MAXC):
            if c + 1 < MAXC:
                issue_idx(c + 1)
            b = c % 2
            outb = outbs[b]
            idx_v = idxbufs[b]
            cid = c * _NW + wid

            @pl.when(cid < NCHUNK)
            def _():
                last = cid == NCHUNK - 1
                goff = jnp.where(last, (LAST_R0 - LAST_SRC) // _L, 0)
                ngrp = jnp.where(last, LAST_G, CH // _L)
                t0 = jnp.where(last, LAST_T0, cid * _CHT)
                if c >= 2:
                    for tr in range(TRC):
                        pltpu.make_async_copy(
                            out_hbm.at[tr, pl.ds(0, _CHT)], outb.at[tr],
                            sems[b][tr]).wait()
                pltpu.make_async_copy(
                    az_hbm.at[pl.ds(0, CH)], idx_v, isems[b]).wait()

                @plsc.parallel_loop(0, ngrp, unroll=2)
                def _(g):
                    z = idx_v[pl.ds((g + goff) * _L, _L)]
                    z = jnp.minimum(jnp.maximum(z, 0), _TBL - 1)
                    cpk = plsc.load_gather(cpk_t, [z])
                    vlo = plsc.load_gather(vlo_t, [z])
                    vhi = plsc.load_gather(vhi_t, [z])
                    clo = cpk & 255
                    chi = cpk >> 8
                    rloc = g * _L
                    t = rloc // 128
                    lst = rloc % 128
                    for cc in range(C):
                        outb[cc // 8, t, cc % 8, pl.ds(lst, _L)] = zeros
                    lvec = lst + iota
                    tvec = jnp.broadcast_to(t, (_L,))
                    plsc.store_scatter(
                        outb, [clo >> 3, tvec, clo & 7, lvec], vlo)
                    plsc.store_scatter(
                        outb, [chi >> 3, tvec, chi & 7, lvec], vhi)

                for tr in range(TRC):
                    pltpu.async_copy(
                        outb.at[tr], out_hbm.at[tr, pl.ds(t0, _CHT)],
                        sems[b][tr])

        for b in range(2):
            for tr in range(TRC):
                pltpu.make_async_copy(
                    out_hbm.at[tr, pl.ds(0, _CHT)], outbs[b].at[tr],
                    sems[b][tr]).wait()

    return encode


def kernel(atomic_numbers, zs):
    n = atomic_numbers.shape[0]
    C = zs.shape[0]
    TRC = -(-C // 8)
    NT = -(-n // 128)
    enc = _make_sc_encode(n, C)
    out4 = enc(atomic_numbers.astype(jnp.int32), zs.astype(jnp.int32))
    return out4.transpose(1, 3, 0, 2).reshape(NT * 128, TRC * 8)[:n, :C]
